# baseline (speedup 1.0000x reference)
import jax
import jax.numpy as jnp
from jax import lax
from jax.experimental import pallas as pl
from jax.experimental.pallas import tpu as pltpu
from jax.experimental.pallas import tpu_sc as plsc

N = 10000
D = 128
E = 320000
NC = 2
NS = 16
NW = NC * NS
EPT = E // NW
CH = 40
NCHUNK = EPT // CH
NBUF = 5
GLEAD = 1
ILEAD = 2
SLAG = 3
ROWS_T0 = 640
ROWS_T15 = N - (NS - 1) * ROWS_T0
ZCH = 40
RCH = 80


def _seg_body(x_hbm, src_hbm, dst_hbm, ew_hbm, psum_hbm, pcnt_hbm,
              sbuf_v, dbuf_v, wbuf_v, rows_v, ones_v, zc_v, accf_s, accc_s,
              gsem, ssem, csem, isem, dsem, wsem):
    c = lax.axis_index("c")
    s = lax.axis_index("s")
    wid = c * NS + s
    base0 = wid * EPT

    def idx_load(j, b):
        base = base0 + j * CH
        pltpu.async_copy(src_hbm.at[pl.ds(base, CH)], sbuf_v.at[b],
                         isem.at[b])
        pltpu.async_copy(dst_hbm.at[pl.ds(base, CH)], dbuf_v.at[b],
                         dsem.at[b])
        pltpu.async_copy(ew_hbm.at[pl.ds(base, CH)], wbuf_v.at[b],
                         wsem.at[b])

    def idx_wait(b):
        pltpu.make_async_copy(src_hbm.at[pl.ds(0, CH)], sbuf_v.at[b],
                              isem.at[b]).wait()
        pltpu.make_async_copy(dst_hbm.at[pl.ds(0, CH)], dbuf_v.at[b],
                              dsem.at[b]).wait()
        pltpu.make_async_copy(ew_hbm.at[pl.ds(0, CH)], wbuf_v.at[b],
                              wsem.at[b]).wait()

    def gather(j, b):
        pltpu.async_copy(x_hbm.at[sbuf_v.at[b]], rows_v.at[b], gsem.at[b])

    for k in range(ILEAD):
        idx_load(k, k)
    for k in range(GLEAD):
        idx_wait(k)
        gather(k, k)

    def zrow_last(t, carry):
        i = t // (D // 16)
        f = t % (D // 16)
        rows_v[NBUF - 1, i, pl.ds(f * 16, 16)] = jnp.zeros((16,), jnp.float32)
        return carry

    lax.fori_loop(0, CH * (D // 16), zrow_last, 0)

    def zsmall(t, carry):
        zc_v[pl.ds(t * 16, 16)] = jnp.zeros((16,), jnp.float32)
        return carry

    lax.fori_loop(0, RCH // 16, zsmall, 0)

    for off in (0, 16, CH - 16):
        ones_v[pl.ds(off, 16)] = jnp.ones((16,), jnp.float32)

    r0 = s * ROWS_T0
    nz = jnp.where(s == NS - 1, ROWS_T15 // ZCH, ROWS_T0 // ZCH)

    def zacc(k, carry):
        r = r0 + k * ZCH
        pltpu.sync_copy(rows_v.at[NBUF - 1], accf_s.at[pl.ds(r, ZCH)])
        return carry

    lax.fori_loop(0, nz, zacc, 0)

    nzc = jnp.where(s == NS - 1, ROWS_T15 // RCH, ROWS_T0 // RCH)

    def zcnt(k, carry):
        r = r0 + k * RCH
        pltpu.sync_copy(zc_v, accc_s.at[pl.ds(r, RCH)])
        return carry

    lax.fori_loop(0, nzc, zcnt, 0)
    plsc.subcore_barrier()

    def ring(g, carry):
        for b in range(NBUF):
            j = g * NBUF + b

            @pl.when(j >= SLAG)
            def _():
                br = (b - SLAG) % NBUF
                pltpu.make_async_copy(
                    rows_v.at[br], accf_s.at[dbuf_v.at[br]],
                    ssem.at[br]).wait()
                pltpu.make_async_copy(
                    ones_v, accc_s.at[dbuf_v.at[br]], csem.at[br]).wait()

            @pl.when(j + ILEAD < NCHUNK)
            def _():
                idx_load(j + ILEAD, (b + ILEAD) % NBUF)

            @pl.when(j + GLEAD < NCHUNK)
            def _():
                bgg = (b + GLEAD) % NBUF
                idx_wait(bgg)
                gather(j + GLEAD, bgg)

            pltpu.make_async_copy(
                x_hbm.at[sbuf_v.at[b]], rows_v.at[b], gsem.at[b]).wait()

            wref = wbuf_v.at[b]

            @plsc.parallel_loop(0, CH, step=1, unroll=4)
            def _(e):
                wsp = plsc.load_gather(
                    wref, [jnp.zeros((16,), jnp.int32) + e])
                for f in range(D // 16):
                    sl = pl.ds(f * 16, 16)
                    rows_v[b, e, sl] = rows_v[b, e, sl] * wsp

            pltpu.async_copy(rows_v.at[b], accf_s.at[dbuf_v.at[b]],
                             ssem.at[b], add=True)
            pltpu.async_copy(ones_v, accc_s.at[dbuf_v.at[b]],
                             csem.at[b], add=True)
        return carry

    lax.fori_loop(0, NCHUNK // NBUF, ring, 0)

    for b in range((NCHUNK - SLAG) % NBUF, (NCHUNK - SLAG) % NBUF + SLAG):
        bb = b % NBUF
        pltpu.make_async_copy(
            rows_v.at[bb], accf_s.at[dbuf_v.at[bb]], ssem.at[bb]).wait()
        pltpu.make_async_copy(
            ones_v, accc_s.at[dbuf_v.at[bb]], csem.at[bb]).wait()
    plsc.subcore_barrier()

    def cout(k, carry):
        r = r0 + k * RCH
        pltpu.sync_copy(accf_s.at[pl.ds(r, RCH)], psum_hbm.at[c, pl.ds(r, RCH)])
        pltpu.sync_copy(accc_s.at[pl.ds(r, RCH)], zc_v)
        pltpu.sync_copy(zc_v, pcnt_hbm.at[pl.ds(c * N + r, RCH)])
        return carry

    lax.fori_loop(0, nzc, cout, 0)


_seg_kernel = pl.kernel(
    _seg_body,
    out_type=(
        jax.ShapeDtypeStruct((NC, N, D), jnp.float32),
        jax.ShapeDtypeStruct((NC * N,), jnp.float32),
    ),
    mesh=plsc.VectorSubcoreMesh(core_axis_name="c", subcore_axis_name="s"),
    compiler_params=pltpu.CompilerParams(needs_layout_passes=False),
    scratch_types=[
        pltpu.VMEM((NBUF, CH), jnp.int32),
        pltpu.VMEM((NBUF, CH), jnp.int32),
        pltpu.VMEM((NBUF, CH), jnp.float32),
        pltpu.VMEM((NBUF, CH, D), jnp.float32),
        pltpu.VMEM((CH,), jnp.float32),
        pltpu.VMEM((RCH,), jnp.float32),
        pltpu.VMEM_SHARED((N, D), jnp.float32),
        pltpu.VMEM_SHARED((N,), jnp.float32),
        pltpu.SemaphoreType.DMA((NBUF,)),
        pltpu.SemaphoreType.DMA((NBUF,)),
        pltpu.SemaphoreType.DMA((NBUF,)),
        pltpu.SemaphoreType.DMA((NBUF,)),
        pltpu.SemaphoreType.DMA((NBUF,)),
        pltpu.SemaphoreType.DMA((NBUF,)),
    ],
)


BN_BLOCK = 2000


def _mm_body(psum_ref, pcnt_ref, w_ref, b_ref, out_ref):
    tot = psum_ref[0] + psum_ref[1]
    cnt = pcnt_ref[0] + pcnt_ref[1]
    mean = jnp.where(cnt > 0.0, tot / jnp.maximum(cnt, 1.0), 0.0)
    y = jnp.dot(mean, w_ref[...], preferred_element_type=jnp.float32)
    out_ref[...] = jnp.maximum(y + b_ref[...], 0.0)


_mm_kernel = pl.pallas_call(
    _mm_body,
    grid=(N // BN_BLOCK,),
    in_specs=[
        pl.BlockSpec((NC, BN_BLOCK, D), lambda i: (0, i, 0)),
        pl.BlockSpec((NC, BN_BLOCK, 1), lambda i: (0, i, 0)),
        pl.BlockSpec((D, D), lambda i: (0, 0)),
        pl.BlockSpec((1, D), lambda i: (0, 0)),
    ],
    out_specs=pl.BlockSpec((BN_BLOCK, D), lambda i: (i, 0)),
    out_shape=jax.ShapeDtypeStruct((N, D), jnp.float32),
)


@jax.jit
def kernel(node_features, edges, edge_weights, gamma, beta, moving_mean,
           moving_var, W, b):
    psum, pcnt = _seg_kernel(node_features, edges[1], edges[0], edge_weights)
    scale = gamma * lax.rsqrt(moving_var + 1e-3)
    Wp = W * scale[:, None]
    bp = (beta - moving_mean * scale) @ W + b
    return _mm_kernel(psum, pcnt.reshape(NC, N, 1), Wp, bp.reshape(1, D))


# speedup vs baseline: 1.0749x; 1.0749x over previous
import jax
import jax.numpy as jnp
from jax import lax
from jax.experimental import pallas as pl
from jax.experimental.pallas import tpu as pltpu
from jax.experimental.pallas import tpu_sc as plsc

N = 10000
D = 128
E = 320000
NC = 2
NS = 16
NW = NC * NS
EPT = E // NW
CH = 40
NCHUNK = EPT // CH
NBUF = 5
GLEAD = 3
ILEAD = 4
SLAG = 1
ROWS_T0 = 640
ROWS_T15 = N - (NS - 1) * ROWS_T0
ZCH = 40
RCH = 80


def _seg_body(x_hbm, src_hbm, dst_hbm, ew_hbm, psum_hbm, pcnt_hbm,
              sbuf_v, dbuf_v, wbuf_v, rows_v, ones_v, zc_v, accf_s, accc_s,
              gsem, ssem, csem, isem, dsem, wsem):
    c = lax.axis_index("c")
    s = lax.axis_index("s")
    wid = c * NS + s
    base0 = wid * EPT

    def idx_load(j, b):
        base = base0 + j * CH
        pltpu.async_copy(src_hbm.at[pl.ds(base, CH)], sbuf_v.at[b],
                         isem.at[b])
        pltpu.async_copy(dst_hbm.at[pl.ds(base, CH)], dbuf_v.at[b],
                         dsem.at[b])
        pltpu.async_copy(ew_hbm.at[pl.ds(base, CH)], wbuf_v.at[b],
                         wsem.at[b])

    def idx_wait(b):
        pltpu.make_async_copy(src_hbm.at[pl.ds(0, CH)], sbuf_v.at[b],
                              isem.at[b]).wait()
        pltpu.make_async_copy(dst_hbm.at[pl.ds(0, CH)], dbuf_v.at[b],
                              dsem.at[b]).wait()
        pltpu.make_async_copy(ew_hbm.at[pl.ds(0, CH)], wbuf_v.at[b],
                              wsem.at[b]).wait()

    def gather(j, b):
        pltpu.async_copy(x_hbm.at[sbuf_v.at[b]], rows_v.at[b], gsem.at[b])

    for k in range(ILEAD):
        idx_load(k, k)
    for k in range(GLEAD):
        idx_wait(k)
        gather(k, k)

    def zrow_last(t, carry):
        i = t // (D // 16)
        f = t % (D // 16)
        rows_v[NBUF - 1, i, pl.ds(f * 16, 16)] = jnp.zeros((16,), jnp.float32)
        return carry

    lax.fori_loop(0, CH * (D // 16), zrow_last, 0)

    def zsmall(t, carry):
        zc_v[pl.ds(t * 16, 16)] = jnp.zeros((16,), jnp.float32)
        return carry

    lax.fori_loop(0, RCH // 16, zsmall, 0)

    for off in (0, 16, CH - 16):
        ones_v[pl.ds(off, 16)] = jnp.ones((16,), jnp.float32)

    r0 = s * ROWS_T0
    nz = jnp.where(s == NS - 1, ROWS_T15 // ZCH, ROWS_T0 // ZCH)

    def zacc(k, carry):
        r = r0 + k * ZCH
        pltpu.sync_copy(rows_v.at[NBUF - 1], accf_s.at[pl.ds(r, ZCH)])
        return carry

    lax.fori_loop(0, nz, zacc, 0)

    nzc = jnp.where(s == NS - 1, ROWS_T15 // RCH, ROWS_T0 // RCH)

    def zcnt(k, carry):
        r = r0 + k * RCH
        pltpu.sync_copy(zc_v, accc_s.at[pl.ds(r, RCH)])
        return carry

    lax.fori_loop(0, nzc, zcnt, 0)
    plsc.subcore_barrier()

    def ring(g, carry):
        for b in range(NBUF):
            j = g * NBUF + b

            @pl.when(j >= SLAG)
            def _():
                br = (b - SLAG) % NBUF
                pltpu.make_async_copy(
                    rows_v.at[br], accf_s.at[dbuf_v.at[br]],
                    ssem.at[br]).wait()
                pltpu.make_async_copy(
                    ones_v, accc_s.at[dbuf_v.at[br]], csem.at[br]).wait()

            @pl.when(j + ILEAD < NCHUNK)
            def _():
                idx_load(j + ILEAD, (b + ILEAD) % NBUF)

            @pl.when(j + GLEAD < NCHUNK)
            def _():
                bgg = (b + GLEAD) % NBUF
                idx_wait(bgg)
                gather(j + GLEAD, bgg)

            pltpu.make_async_copy(
                x_hbm.at[sbuf_v.at[b]], rows_v.at[b], gsem.at[b]).wait()

            wref = wbuf_v.at[b]

            @plsc.parallel_loop(0, CH, step=1, unroll=4)
            def _(e):
                wsp = plsc.load_gather(
                    wref, [jnp.zeros((16,), jnp.int32) + e])
                for f in range(D // 16):
                    sl = pl.ds(f * 16, 16)
                    rows_v[b, e, sl] = rows_v[b, e, sl] * wsp

            pltpu.async_copy(rows_v.at[b], accf_s.at[dbuf_v.at[b]],
                             ssem.at[b], add=True)
            pltpu.async_copy(ones_v, accc_s.at[dbuf_v.at[b]],
                             csem.at[b], add=True)
        return carry

    lax.fori_loop(0, NCHUNK // NBUF, ring, 0)

    for b in range((NCHUNK - SLAG) % NBUF, (NCHUNK - SLAG) % NBUF + SLAG):
        bb = b % NBUF
        pltpu.make_async_copy(
            rows_v.at[bb], accf_s.at[dbuf_v.at[bb]], ssem.at[bb]).wait()
        pltpu.make_async_copy(
            ones_v, accc_s.at[dbuf_v.at[bb]], csem.at[bb]).wait()
    plsc.subcore_barrier()

    def cout(k, carry):
        r = r0 + k * RCH
        pltpu.sync_copy(accf_s.at[pl.ds(r, RCH)], psum_hbm.at[c, pl.ds(r, RCH)])
        pltpu.sync_copy(accc_s.at[pl.ds(r, RCH)], zc_v)
        pltpu.sync_copy(zc_v, pcnt_hbm.at[pl.ds(c * N + r, RCH)])
        return carry

    lax.fori_loop(0, nzc, cout, 0)


_seg_kernel = pl.kernel(
    _seg_body,
    out_type=(
        jax.ShapeDtypeStruct((NC, N, D), jnp.float32),
        jax.ShapeDtypeStruct((NC * N,), jnp.float32),
    ),
    mesh=plsc.VectorSubcoreMesh(core_axis_name="c", subcore_axis_name="s"),
    compiler_params=pltpu.CompilerParams(needs_layout_passes=False),
    scratch_types=[
        pltpu.VMEM((NBUF, CH), jnp.int32),
        pltpu.VMEM((NBUF, CH), jnp.int32),
        pltpu.VMEM((NBUF, CH), jnp.float32),
        pltpu.VMEM((NBUF, CH, D), jnp.float32),
        pltpu.VMEM((CH,), jnp.float32),
        pltpu.VMEM((RCH,), jnp.float32),
        pltpu.VMEM_SHARED((N, D), jnp.float32),
        pltpu.VMEM_SHARED((N,), jnp.float32),
        pltpu.SemaphoreType.DMA((NBUF,)),
        pltpu.SemaphoreType.DMA((NBUF,)),
        pltpu.SemaphoreType.DMA((NBUF,)),
        pltpu.SemaphoreType.DMA((NBUF,)),
        pltpu.SemaphoreType.DMA((NBUF,)),
        pltpu.SemaphoreType.DMA((NBUF,)),
    ],
)


BN_BLOCK = 2000


def _mm_body(psum_ref, pcnt_ref, w_ref, b_ref, out_ref):
    tot = psum_ref[0] + psum_ref[1]
    cnt = pcnt_ref[0] + pcnt_ref[1]
    mean = jnp.where(cnt > 0.0, tot / jnp.maximum(cnt, 1.0), 0.0)
    y = jnp.dot(mean, w_ref[...], preferred_element_type=jnp.float32)
    out_ref[...] = jnp.maximum(y + b_ref[...], 0.0)


_mm_kernel = pl.pallas_call(
    _mm_body,
    grid=(N // BN_BLOCK,),
    in_specs=[
        pl.BlockSpec((NC, BN_BLOCK, D), lambda i: (0, i, 0)),
        pl.BlockSpec((NC, BN_BLOCK, 1), lambda i: (0, i, 0)),
        pl.BlockSpec((D, D), lambda i: (0, 0)),
        pl.BlockSpec((1, D), lambda i: (0, 0)),
    ],
    out_specs=pl.BlockSpec((BN_BLOCK, D), lambda i: (i, 0)),
    out_shape=jax.ShapeDtypeStruct((N, D), jnp.float32),
)


@jax.jit
def kernel(node_features, edges, edge_weights, gamma, beta, moving_mean,
           moving_var, W, b):
    psum, pcnt = _seg_kernel(node_features, edges[1], edges[0], edge_weights)
    scale = gamma * lax.rsqrt(moving_var + 1e-3)
    Wp = W * scale[:, None]
    bp = (beta - moving_mean * scale) @ W + b
    return _mm_kernel(psum, pcnt.reshape(NC, N, 1), Wp, bp.reshape(1, D))


# speedup vs baseline: 1.2653x; 1.1772x over previous
import jax
import jax.numpy as jnp
from jax import lax
from jax.experimental import pallas as pl
from jax.experimental.pallas import tpu as pltpu
from jax.experimental.pallas import tpu_sc as plsc

N = 10000
D = 128
E = 320000
NC = 2
NS = 16
NW = NC * NS
EPT = E // NW
CH = 40
NCHUNK = EPT // CH
NBUF = 5
GLEAD = 2
ILEAD = 3
SLAG = 2
ROWS_T0 = 640
ROWS_T15 = N - (NS - 1) * ROWS_T0
ZCH = 40
RCH = 80


def _seg_body(x_hbm, ef_hbm, ew_hbm, psum_hbm, pcnt_hbm,
              sbuf_v, dbuf_v, wbuf_v, rows_v, ones_v, zc_v, accf_s, accc_s,
              gsem, ssem, csem, isem, dsem, wsem):
    c = lax.axis_index("c")
    s = lax.axis_index("s")
    wid = c * NS + s
    base0 = wid * EPT

    def idx_load(j, b):
        base = base0 + j * CH
        pltpu.async_copy(ef_hbm.at[pl.ds(E + base, CH)], sbuf_v.at[b],
                         isem.at[b])
        pltpu.async_copy(ef_hbm.at[pl.ds(base, CH)], dbuf_v.at[b],
                         dsem.at[b])
        pltpu.async_copy(ew_hbm.at[pl.ds(base, CH)], wbuf_v.at[b],
                         wsem.at[b])

    def idx_wait(b):
        pltpu.make_async_copy(ef_hbm.at[pl.ds(0, CH)], sbuf_v.at[b],
                              isem.at[b]).wait()
        pltpu.make_async_copy(ef_hbm.at[pl.ds(0, CH)], dbuf_v.at[b],
                              dsem.at[b]).wait()
        pltpu.make_async_copy(ew_hbm.at[pl.ds(0, CH)], wbuf_v.at[b],
                              wsem.at[b]).wait()

    def gather(j, b):
        pltpu.async_copy(x_hbm.at[sbuf_v.at[b]], rows_v.at[b], gsem.at[b])

    for k in range(ILEAD):
        idx_load(k, k)
    for k in range(GLEAD):
        idx_wait(k)
        gather(k, k)

    def zrow_last(t, carry):
        i = t // (D // 16)
        f = t % (D // 16)
        rows_v[NBUF - 1, i, pl.ds(f * 16, 16)] = jnp.zeros((16,), jnp.float32)
        return carry

    lax.fori_loop(0, CH * (D // 16), zrow_last, 0)

    def zsmall(t, carry):
        zc_v[pl.ds(t * 16, 16)] = jnp.zeros((16,), jnp.float32)
        return carry

    lax.fori_loop(0, RCH // 16, zsmall, 0)

    for off in (0, 16, CH - 16):
        ones_v[pl.ds(off, 16)] = jnp.ones((16,), jnp.float32)

    r0 = s * ROWS_T0
    nz = jnp.where(s == NS - 1, ROWS_T15 // ZCH, ROWS_T0 // ZCH)

    def zacc(k, carry):
        r = r0 + k * ZCH
        pltpu.sync_copy(rows_v.at[NBUF - 1], accf_s.at[pl.ds(r, ZCH)])
        return carry

    lax.fori_loop(0, nz, zacc, 0)

    nzc = jnp.where(s == NS - 1, ROWS_T15 // RCH, ROWS_T0 // RCH)

    def zcnt(k, carry):
        r = r0 + k * RCH
        pltpu.sync_copy(zc_v, accc_s.at[pl.ds(r, RCH)])
        return carry

    lax.fori_loop(0, nzc, zcnt, 0)
    plsc.subcore_barrier()

    def ring(g, carry):
        for b in range(NBUF):
            j = g * NBUF + b

            @pl.when(j >= SLAG)
            def _():
                br = (b - SLAG) % NBUF
                pltpu.make_async_copy(
                    rows_v.at[br], accf_s.at[dbuf_v.at[br]],
                    ssem.at[br]).wait()
                pltpu.make_async_copy(
                    ones_v, accc_s.at[dbuf_v.at[br]], csem.at[br]).wait()

            @pl.when(j + ILEAD < NCHUNK)
            def _():
                idx_load(j + ILEAD, (b + ILEAD) % NBUF)

            @pl.when(j + GLEAD < NCHUNK)
            def _():
                bgg = (b + GLEAD) % NBUF
                idx_wait(bgg)
                gather(j + GLEAD, bgg)

            pltpu.make_async_copy(
                x_hbm.at[sbuf_v.at[b]], rows_v.at[b], gsem.at[b]).wait()

            wref = wbuf_v.at[b]

            @plsc.parallel_loop(0, CH, step=1, unroll=4)
            def _(e):
                wsp = plsc.load_gather(
                    wref, [jnp.zeros((16,), jnp.int32) + e])
                for f in range(D // 16):
                    sl = pl.ds(f * 16, 16)
                    rows_v[b, e, sl] = rows_v[b, e, sl] * wsp

            pltpu.async_copy(rows_v.at[b], accf_s.at[dbuf_v.at[b]],
                             ssem.at[b], add=True)
            pltpu.async_copy(ones_v, accc_s.at[dbuf_v.at[b]],
                             csem.at[b], add=True)
        return carry

    lax.fori_loop(0, NCHUNK // NBUF, ring, 0)

    for b in range((NCHUNK - SLAG) % NBUF, (NCHUNK - SLAG) % NBUF + SLAG):
        bb = b % NBUF
        pltpu.make_async_copy(
            rows_v.at[bb], accf_s.at[dbuf_v.at[bb]], ssem.at[bb]).wait()
        pltpu.make_async_copy(
            ones_v, accc_s.at[dbuf_v.at[bb]], csem.at[bb]).wait()
    plsc.subcore_barrier()

    def cout(k, carry):
        r = r0 + k * RCH
        pltpu.sync_copy(accf_s.at[pl.ds(r, RCH)], psum_hbm.at[c, pl.ds(r, RCH)])
        pltpu.sync_copy(accc_s.at[pl.ds(r, RCH)], zc_v)
        pltpu.sync_copy(zc_v, pcnt_hbm.at[pl.ds(c * N + r, RCH)])
        return carry

    lax.fori_loop(0, nzc, cout, 0)


_seg_kernel = pl.kernel(
    _seg_body,
    out_type=(
        jax.ShapeDtypeStruct((NC, N, D), jnp.float32),
        jax.ShapeDtypeStruct((NC * N,), jnp.float32),
    ),
    mesh=plsc.VectorSubcoreMesh(core_axis_name="c", subcore_axis_name="s"),
    compiler_params=pltpu.CompilerParams(needs_layout_passes=False),
    scratch_types=[
        pltpu.VMEM((NBUF, CH), jnp.int32),
        pltpu.VMEM((NBUF, CH), jnp.int32),
        pltpu.VMEM((NBUF, CH), jnp.float32),
        pltpu.VMEM((NBUF, CH, D), jnp.float32),
        pltpu.VMEM((CH,), jnp.float32),
        pltpu.VMEM((RCH,), jnp.float32),
        pltpu.VMEM_SHARED((N, D), jnp.float32),
        pltpu.VMEM_SHARED((N,), jnp.float32),
        pltpu.SemaphoreType.DMA((NBUF,)),
        pltpu.SemaphoreType.DMA((NBUF,)),
        pltpu.SemaphoreType.DMA((NBUF,)),
        pltpu.SemaphoreType.DMA((NBUF,)),
        pltpu.SemaphoreType.DMA((NBUF,)),
        pltpu.SemaphoreType.DMA((NBUF,)),
    ],
)


BN_BLOCK = 2000


def _mm_body(psum_ref, pcnt_ref, w_ref, b_ref, out_ref):
    tot = psum_ref[0] + psum_ref[1]
    cnt = pcnt_ref[0] + pcnt_ref[1]
    mean = jnp.where(cnt > 0.0, tot / jnp.maximum(cnt, 1.0), 0.0)
    y = jnp.dot(mean, w_ref[...], preferred_element_type=jnp.float32)
    out_ref[...] = jnp.maximum(y + b_ref[...], 0.0)


_mm_kernel = pl.pallas_call(
    _mm_body,
    grid=(N // BN_BLOCK,),
    in_specs=[
        pl.BlockSpec((NC, BN_BLOCK, D), lambda i: (0, i, 0)),
        pl.BlockSpec((NC, BN_BLOCK, 1), lambda i: (0, i, 0)),
        pl.BlockSpec((D, D), lambda i: (0, 0)),
        pl.BlockSpec((1, D), lambda i: (0, 0)),
    ],
    out_specs=pl.BlockSpec((BN_BLOCK, D), lambda i: (i, 0)),
    out_shape=jax.ShapeDtypeStruct((N, D), jnp.float32),
)


@jax.jit
def kernel(node_features, edges, edge_weights, gamma, beta, moving_mean,
           moving_var, W, b):
    psum, pcnt = _seg_kernel(node_features, edges.reshape(2 * E),
                             edge_weights)
    scale = gamma * lax.rsqrt(moving_var + 1e-3)
    Wp = W * scale[:, None]
    bp = (beta - moving_mean * scale) @ W + b
    return _mm_kernel(psum, pcnt.reshape(NC, N, 1), Wp, bp.reshape(1, D))
